# baseline (speedup 1.0000x reference)
import functools

import jax
import jax.numpy as jnp
from jax import lax
from jax.experimental import pallas as pl
from jax.experimental.pallas import tpu as pltpu
from jax.experimental.pallas import tpu_sc as plsc

N, E, D = 10000, 320000, 128
DH = D // 2

NC, NS = 2, 16
CH = 128
C1 = 160
CG = 40
NB = 4
EP = NS * C1 * CH
NP = 10112
RPT = NP // NS

_PARAMS = pltpu.CompilerParams(use_tc_tiling_on_sc=False)
_MESH = dict(core_axis_name="c", subcore_axis_name="s")


def _make_sc_agg():
    out_type = [jax.ShapeDtypeStruct((NC, NP, DH), jnp.float32)]
    scratch = [
        pltpu.VMEM((CG, 2, CH), jnp.int32),
        pltpu.VMEM((NB, CH, DH), jnp.float32),
        pltpu.VMEM_SHARED((N, DH), jnp.float32),
        pltpu.VMEM_SHARED((NP, DH), jnp.float32),
    ] + [pltpu.SemaphoreType.DMA] * NB

    def body(h, ecr, zrows, agg_out, idx_v, rows_v, tab_sh, acc_sh, *sems):
        c = lax.axis_index("c")
        s = lax.axis_index("s")
        row0 = s * RPT
        pltpu.sync_copy(zrows, acc_sh.at[pl.ds(row0, RPT)])

        @pl.when(s == 0)
        def _():
            pltpu.sync_copy(h.at[c], tab_sh)
        plsc.subcore_barrier()

        def issue(j, b):
            pltpu.async_copy(tab_sh.at[idx_v.at[j, 0]], rows_v.at[b],
                             sems[b])

        def wait(j, b):
            pltpu.make_async_copy(tab_sh.at[idx_v.at[j, 0]], rows_v.at[b],
                                  sems[b]).wait()

        def scat(j, b):
            pltpu.sync_copy(rows_v.at[b], acc_sh.at[idx_v.at[j, 1]],
                            add=True)

        for g in range(C1 // CG):
            pltpu.sync_copy(ecr.at[s, pl.ds(g * CG, CG)], idx_v)
            for b in range(NB - 1):
                issue(b, b)

            def grp(q, carry):
                for b in range(NB):
                    j = q * NB + b
                    wait(j, b)
                    issue(j + NB - 1, (b + NB - 1) % NB)
                    scat(j, b)
                return carry

            lax.fori_loop(0, CG // NB - 1, grp, 0)
            base = CG - NB
            wait(base, 0)
            issue(CG - 1, NB - 1)
            scat(base, 0)
            for b in range(1, NB):
                wait(base + b, b)
                scat(base + b, b)
        plsc.subcore_barrier()
        pltpu.sync_copy(acc_sh.at[pl.ds(row0, RPT)],
                        agg_out.at[c, pl.ds(row0, RPT)])

    return pl.kernel(body, out_type,
                     mesh=plsc.VectorSubcoreMesh(**_MESH),
                     scratch_types=scratch, compiler_params=_PARAMS)


def _make_sc_deg():
    CD = C1 // NC
    out_type = [jax.ShapeDtypeStruct((NC, NP, 16), jnp.float32)]
    scratch = [
        pltpu.VMEM((CD, CH), jnp.int32),
        pltpu.VMEM((CH, 16), jnp.float32),
        pltpu.VMEM_SHARED((NP, 16), jnp.float32),
    ]

    def body(dstr, zdeg, ones, deg_out, dst_v, ones_v, deg_sh):
        c = lax.axis_index("c")
        s = lax.axis_index("s")
        pltpu.sync_copy(dstr.at[s, pl.ds(c * CD, CD)], dst_v)
        pltpu.sync_copy(ones, ones_v)
        row0 = s * RPT
        pltpu.sync_copy(zdeg, deg_sh.at[pl.ds(row0, RPT)])
        plsc.subcore_barrier()

        def chunk(j, carry):
            pltpu.sync_copy(ones_v, deg_sh.at[dst_v.at[j]], add=True)
            return carry

        lax.fori_loop(0, CD, chunk, 0)
        plsc.subcore_barrier()
        pltpu.sync_copy(deg_sh.at[pl.ds(row0, RPT)],
                        deg_out.at[c, pl.ds(row0, RPT)])

    return pl.kernel(body, out_type,
                     mesh=plsc.VectorSubcoreMesh(**_MESH),
                     scratch_types=scratch, compiler_params=_PARAMS)


_sc_agg_call = _make_sc_agg()
_sc_deg_call = _make_sc_deg()

_BR = 2000


def _tc_body(agg, deg, h, wl, wr, b, out, *, final):
    ssum = jnp.concatenate([agg[0], agg[1]], axis=-1)
    dg = deg[0, :, 0:1] + deg[1, :, 0:1]
    mean = ssum / jnp.maximum(dg, 1.0)
    hcat = jnp.concatenate([h[0], h[1]], axis=-1)
    o = (jnp.dot(mean, wl[...], preferred_element_type=jnp.float32)
         + jnp.dot(hcat, wr[...], preferred_element_type=jnp.float32)
         + b[...])
    if final:
        m = jnp.max(o, axis=-1, keepdims=True)
        lo = o - m
        out[...] = lo - jnp.log(jnp.sum(jnp.exp(lo), axis=-1, keepdims=True))
    else:
        a = jnp.where(o > 0, o, jnp.exp(jnp.minimum(o, 0.0)) - 1.0)
        out[0] = a[:, :DH]
        out[1] = a[:, DH:]


def _tc_layer(agg, deg, h, wl, wr, b, final):
    if final:
        out_spec = pl.BlockSpec((_BR, D), lambda i: (i, 0))
        out_shape = jax.ShapeDtypeStruct((N, D), jnp.float32)
    else:
        out_spec = pl.BlockSpec((NC, _BR, DH), lambda i: (0, i, 0))
        out_shape = jax.ShapeDtypeStruct((NC, N, DH), jnp.float32)
    return pl.pallas_call(
        functools.partial(_tc_body, final=final),
        grid=(N // _BR,),
        in_specs=[
            pl.BlockSpec((NC, _BR, DH), lambda i: (0, i, 0)),
            pl.BlockSpec((NC, _BR, 16), lambda i: (0, i, 0)),
            pl.BlockSpec((NC, _BR, DH), lambda i: (0, i, 0)),
            pl.BlockSpec((D, D), lambda i: (0, 0)),
            pl.BlockSpec((D, D), lambda i: (0, 0)),
            pl.BlockSpec((1, D), lambda i: (0, 0)),
        ],
        out_specs=out_spec,
        out_shape=out_shape,
    )(agg, deg, h, wl, wr, b)


def kernel(x, edge_index, W1l, W1r, b1, W2l, W2r, b2, W3l, W3r, b3):
    pad = EP - E
    src = jnp.concatenate([edge_index[0], jnp.zeros((pad,), jnp.int32)])
    dst = jnp.concatenate([edge_index[1], jnp.full((pad,), N, jnp.int32)])
    ec = jnp.stack([src.reshape(NS, C1, CH), dst.reshape(NS, C1, CH)],
                   axis=2)
    dst1 = dst.reshape(NS, C1, CH)
    zrows = jnp.zeros((RPT, DH), jnp.float32)
    zdeg = jnp.zeros((RPT, 16), jnp.float32)
    ones = jnp.ones((CH, 16), jnp.float32)
    xs = x.reshape(N, NC, DH).transpose(1, 0, 2)

    deg, = _sc_deg_call(dst1, zdeg, ones)
    agg1, = _sc_agg_call(xs, ec, zrows)
    h1 = _tc_layer(agg1, deg, xs, W1l, W1r, b1.reshape(1, D), final=False)
    agg2, = _sc_agg_call(h1, ec, zrows)
    h2 = _tc_layer(agg2, deg, h1, W2l, W2r, b2.reshape(1, D), final=False)
    agg3, = _sc_agg_call(h2, ec, zrows)
    return _tc_layer(agg3, deg, h2, W3l, W3r, b3.reshape(1, D), final=True)


# speedup vs baseline: 1.4888x; 1.4888x over previous
import functools

import jax
import jax.numpy as jnp
from jax import lax
from jax.experimental import pallas as pl
from jax.experimental.pallas import tpu as pltpu
from jax.experimental.pallas import tpu_sc as plsc

N, E, D = 10000, 320000, 128
DH = D // 2

NC, NS = 2, 16
CH = 128
C1 = 160
CG = 80
NB = 2
EP = NS * C1 * CH
NP = 10112
RPT = NP // NS

_PARAMS = pltpu.CompilerParams(use_tc_tiling_on_sc=False)
_MESH = dict(core_axis_name="c", subcore_axis_name="s")


def _make_sc_agg():
    out_type = [jax.ShapeDtypeStruct((NC, NP, DH), jnp.bfloat16)]
    scratch = [
        pltpu.VMEM((CG, 2, CH), jnp.int32),
        pltpu.VMEM((NB, CH, DH), jnp.bfloat16),
        pltpu.VMEM_SHARED((N, DH), jnp.bfloat16),
        pltpu.VMEM_SHARED((NP, DH), jnp.bfloat16),
    ] + [pltpu.SemaphoreType.DMA] * NB

    def body(h, ecr, zrows, agg_out, idx_v, rows_v, tab_sh, acc_sh, *sems):
        c = lax.axis_index("c")
        s = lax.axis_index("s")
        row0 = s * RPT
        pltpu.sync_copy(zrows, acc_sh.at[pl.ds(row0, RPT)])

        @pl.when(s == 0)
        def _():
            pltpu.sync_copy(h.at[c], tab_sh)
        plsc.subcore_barrier()

        def issue(j, b):
            pltpu.async_copy(tab_sh.at[idx_v.at[j, 0]], rows_v.at[b],
                             sems[b])

        def wait(j, b):
            pltpu.make_async_copy(tab_sh.at[idx_v.at[j, 0]], rows_v.at[b],
                                  sems[b]).wait()

        def scat(j, b):
            pltpu.sync_copy(rows_v.at[b], acc_sh.at[idx_v.at[j, 1]],
                            add=True)

        for g in range(C1 // CG):
            pltpu.sync_copy(ecr.at[s, pl.ds(g * CG, CG)], idx_v)
            for b in range(NB - 1):
                issue(b, b)

            def grp(q, carry):
                for b in range(NB):
                    j = q * NB + b
                    wait(j, b)
                    issue(j + NB - 1, (b + NB - 1) % NB)
                    scat(j, b)
                return carry

            lax.fori_loop(0, CG // NB - 1, grp, 0)
            base = CG - NB
            wait(base, 0)
            issue(CG - 1, NB - 1)
            scat(base, 0)
            for b in range(1, NB):
                wait(base + b, b)
                scat(base + b, b)
        plsc.subcore_barrier()
        pltpu.sync_copy(acc_sh.at[pl.ds(row0, RPT)],
                        agg_out.at[c, pl.ds(row0, RPT)])

    return pl.kernel(body, out_type,
                     mesh=plsc.VectorSubcoreMesh(**_MESH),
                     scratch_types=scratch, compiler_params=_PARAMS)


def _make_sc_deg():
    CD = C1 // NC
    out_type = [jax.ShapeDtypeStruct((NC, NP, 16), jnp.float32)]
    scratch = [
        pltpu.VMEM((CD, CH), jnp.int32),
        pltpu.VMEM((CH, 16), jnp.float32),
        pltpu.VMEM_SHARED((NP, 16), jnp.float32),
    ]

    def body(dstr, zdeg, ones, deg_out, dst_v, ones_v, deg_sh):
        c = lax.axis_index("c")
        s = lax.axis_index("s")
        pltpu.sync_copy(dstr.at[s, pl.ds(c * CD, CD)], dst_v)
        pltpu.sync_copy(ones, ones_v)
        row0 = s * RPT
        pltpu.sync_copy(zdeg, deg_sh.at[pl.ds(row0, RPT)])
        plsc.subcore_barrier()

        def chunk(j, carry):
            pltpu.sync_copy(ones_v, deg_sh.at[dst_v.at[j]], add=True)
            return carry

        lax.fori_loop(0, CD, chunk, 0)
        plsc.subcore_barrier()
        pltpu.sync_copy(deg_sh.at[pl.ds(row0, RPT)],
                        deg_out.at[c, pl.ds(row0, RPT)])

    return pl.kernel(body, out_type,
                     mesh=plsc.VectorSubcoreMesh(**_MESH),
                     scratch_types=scratch, compiler_params=_PARAMS)


_sc_agg_call = _make_sc_agg()
_sc_deg_call = _make_sc_deg()

_BR = 2000


def _tc_body(agg, deg, h, wl, wr, b, *outs, final):
    ssum = jnp.concatenate([agg[0], agg[1]], axis=-1).astype(jnp.float32)
    dg = deg[0, :, 0:1] + deg[1, :, 0:1]
    mean = ssum / jnp.maximum(dg, 1.0)
    hcat = jnp.concatenate([h[0], h[1]], axis=-1)
    o = (jnp.dot(mean, wl[...], preferred_element_type=jnp.float32)
         + jnp.dot(hcat, wr[...], preferred_element_type=jnp.float32)
         + b[...])
    if final:
        m = jnp.max(o, axis=-1, keepdims=True)
        lo = o - m
        outs[0][...] = (lo - jnp.log(jnp.sum(jnp.exp(lo), axis=-1,
                                             keepdims=True)))
    else:
        a = jnp.where(o > 0, o, jnp.exp(jnp.minimum(o, 0.0)) - 1.0)
        out, outb = outs
        out[0] = a[:, :DH]
        out[1] = a[:, DH:]
        ab = a.astype(jnp.bfloat16)
        outb[0] = ab[:, :DH]
        outb[1] = ab[:, DH:]


def _tc_layer(agg, deg, h, wl, wr, b, final):
    if final:
        out_spec = pl.BlockSpec((_BR, D), lambda i: (i, 0))
        out_shape = jax.ShapeDtypeStruct((N, D), jnp.float32)
    else:
        out_spec = [pl.BlockSpec((NC, _BR, DH), lambda i: (0, i, 0)),
                    pl.BlockSpec((NC, _BR, DH), lambda i: (0, i, 0))]
        out_shape = [jax.ShapeDtypeStruct((NC, N, DH), jnp.float32),
                     jax.ShapeDtypeStruct((NC, N, DH), jnp.bfloat16)]
    return pl.pallas_call(
        functools.partial(_tc_body, final=final),
        grid=(N // _BR,),
        in_specs=[
            pl.BlockSpec((NC, _BR, DH), lambda i: (0, i, 0)),
            pl.BlockSpec((NC, _BR, 16), lambda i: (0, i, 0)),
            pl.BlockSpec((NC, _BR, DH), lambda i: (0, i, 0)),
            pl.BlockSpec((D, D), lambda i: (0, 0)),
            pl.BlockSpec((D, D), lambda i: (0, 0)),
            pl.BlockSpec((1, D), lambda i: (0, 0)),
        ],
        out_specs=out_spec,
        out_shape=out_shape,
    )(agg, deg, h, wl, wr, b)


def kernel(x, edge_index, W1l, W1r, b1, W2l, W2r, b2, W3l, W3r, b3):
    pad = EP - E
    src = jnp.concatenate([edge_index[0], jnp.zeros((pad,), jnp.int32)])
    dst = jnp.concatenate([edge_index[1], jnp.full((pad,), N, jnp.int32)])
    ec = jnp.stack([src.reshape(NS, C1, CH), dst.reshape(NS, C1, CH)],
                   axis=2)
    dst1 = dst.reshape(NS, C1, CH)
    zrows = jnp.zeros((RPT, DH), jnp.bfloat16)
    zdeg = jnp.zeros((RPT, 16), jnp.float32)
    ones = jnp.ones((CH, 16), jnp.float32)
    xs = x.reshape(N, NC, DH).transpose(1, 0, 2)
    xsb = xs.astype(jnp.bfloat16)

    deg, = _sc_deg_call(dst1, zdeg, ones)
    agg1, = _sc_agg_call(xsb, ec, zrows)
    h1, h1b = _tc_layer(agg1, deg, xs, W1l, W1r, b1.reshape(1, D),
                        final=False)
    agg2, = _sc_agg_call(h1b, ec, zrows)
    h2, h2b = _tc_layer(agg2, deg, h1, W2l, W2r, b2.reshape(1, D),
                        final=False)
    agg3, = _sc_agg_call(h2b, ec, zrows)
    return _tc_layer(agg3, deg, h2, W3l, W3r, b3.reshape(1, D), final=True)
